# Initial kernel scaffold; baseline (speedup 1.0000x reference)
import jax
import jax.numpy as jnp
from jax.experimental import pallas as pl


def kernel(x, edge_index, W1_l, W1_r, b1, W2_l, W2_r, b2):
    raise NotImplementedError("write your pallas kernel here")


# speedup vs baseline: 4.7601x; 4.7601x over previous
import functools

import jax
import jax.numpy as jnp
from jax import lax
from jax.experimental import pallas as pl
from jax.experimental.pallas import tpu as pltpu
from jax.experimental.pallas import tpu_sc as plsc

N_NODES = 10000
N_EDGES = 320000
D_IN = 128
D_HID = 128
N_CLS = 16

NC = 2
NS = 16
NW = NC * NS
CHUNK = 128
CH_PER_W = 80
EP = NW * CH_PER_W * CHUNK
NP = NS * 640
RPT = NP // NS

BLK = 128
GRID = NP // BLK


def _make_agg(d, with_cnt):
  out_type = [jax.ShapeDtypeStruct((NC, NP, d), jnp.float32)]
  if with_cnt:
    out_type.append(jax.ShapeDtypeStruct((NC, NP), jnp.float32))

  scratch = [
      pltpu.VMEM((CH_PER_W, CHUNK), jnp.int32),
      pltpu.VMEM((CH_PER_W, CHUNK), jnp.int32),
      pltpu.VMEM((CHUNK, d), jnp.float32),
      pltpu.VMEM_SHARED((NP, d), jnp.float32),
      pltpu.SemaphoreType.DMA,
  ]
  if with_cnt:
    scratch += [
        pltpu.VMEM((CHUNK,), jnp.float32),
        pltpu.VMEM((RPT,), jnp.float32),
        pltpu.VMEM_SHARED((NP,), jnp.float32),
    ]

  mesh = plsc.VectorSubcoreMesh(core_axis_name="c", subcore_axis_name="s")

  @functools.partial(pl.kernel, mesh=mesh, out_type=out_type,
                     scratch_types=scratch,
                     compiler_params=pltpu.CompilerParams(
                         use_tc_tiling_on_sc=False))
  def body(feat_hbm, src_hbm, dst_hbm, *rest):
    if with_cnt:
      (agg_out, cnt_out, src_v, dst_v, rows_v, agg_sh, sem,
       ones_v, zc_v, cnt_sh) = rest
    else:
      agg_out, src_v, dst_v, rows_v, agg_sh, sem = rest

    cid = lax.axis_index("c")
    sid = lax.axis_index("s")
    wid = cid * NS + sid

    pltpu.sync_copy(src_hbm.at[wid], src_v)
    pltpu.sync_copy(dst_hbm.at[wid], dst_v)

    z16 = jnp.zeros((16,), jnp.float32)
    g = d // 16

    def zrow(i, c):
      rows_v[i // g, pl.ds((i % g) * 16, 16)] = z16
      return c
    lax.fori_loop(0, CHUNK * g, zrow, 0)
    for k in range(RPT // CHUNK):
      pltpu.sync_copy(rows_v,
                      agg_sh.at[pl.ds(sid * RPT + k * CHUNK, CHUNK)])

    if with_cnt:
      one16 = jnp.ones((16,), jnp.float32)
      for k in range(CHUNK // 16):
        ones_v[pl.ds(k * 16, 16)] = one16

      def zcnt(i, c):
        zc_v[pl.ds(i * 16, 16)] = z16
        return c
      lax.fori_loop(0, RPT // 16, zcnt, 0)
      pltpu.sync_copy(zc_v, cnt_sh.at[pl.ds(sid * RPT, RPT)])

    plsc.subcore_barrier()

    def chunk_body(j, c):
      pltpu.async_copy(feat_hbm.at[src_v.at[j]], rows_v, sem).wait()
      pltpu.sync_copy(rows_v, agg_sh.at[dst_v.at[j]], add=True)
      if with_cnt:
        pltpu.sync_copy(ones_v, cnt_sh.at[dst_v.at[j]], add=True)
      return c
    lax.fori_loop(0, CH_PER_W, chunk_body, 0)

    plsc.subcore_barrier()

    r0 = sid * RPT
    pltpu.sync_copy(agg_sh.at[pl.ds(r0, RPT)],
                    agg_out.at[cid, pl.ds(r0, RPT)])
    if with_cnt:
      pltpu.sync_copy(cnt_sh.at[pl.ds(r0, RPT)],
                      cnt_out.at[cid, pl.ds(r0, RPT)])

  return body


_agg_l1 = _make_agg(D_IN, with_cnt=True)
_agg_l2 = _make_agg(N_CLS, with_cnt=False)


def _tc1_body(aggp, cnt_t, xp, w1l, w1r, b1, w2l, w2r, b2,
              h_out, y2_out, z2_out):
  agg = aggp[0] + aggp[1]
  cnt = cnt_t[:, 0] + cnt_t[:, 1]
  inv = 1.0 / jnp.maximum(cnt, 1.0)
  mean = agg * inv[:, None]
  h = mean @ w1l[...] + xp[...] @ w1r[...] + b1[...]
  h = jnp.maximum(h, 0.0)
  h_out[...] = h
  y2_out[...] = h @ w2l[...]
  z2_out[...] = h @ w2r[...] + b2[...]


_tc1 = pl.pallas_call(
    _tc1_body,
    grid=(GRID,),
    in_specs=[
        pl.BlockSpec((NC, BLK, D_IN), lambda i: (0, i, 0)),
        pl.BlockSpec((BLK, NC), lambda i: (i, 0)),
        pl.BlockSpec((BLK, D_IN), lambda i: (i, 0)),
        pl.BlockSpec((D_IN, D_HID), lambda i: (0, 0)),
        pl.BlockSpec((D_IN, D_HID), lambda i: (0, 0)),
        pl.BlockSpec((1, D_HID), lambda i: (0, 0)),
        pl.BlockSpec((D_HID, N_CLS), lambda i: (0, 0)),
        pl.BlockSpec((D_HID, N_CLS), lambda i: (0, 0)),
        pl.BlockSpec((1, N_CLS), lambda i: (0, 0)),
    ],
    out_specs=[
        pl.BlockSpec((BLK, D_HID), lambda i: (i, 0)),
        pl.BlockSpec((BLK, N_CLS), lambda i: (i, 0)),
        pl.BlockSpec((BLK, N_CLS), lambda i: (i, 0)),
    ],
    out_shape=[
        jax.ShapeDtypeStruct((NP, D_HID), jnp.float32),
        jax.ShapeDtypeStruct((NP, N_CLS), jnp.float32),
        jax.ShapeDtypeStruct((NP, N_CLS), jnp.float32),
    ],
)


def _tc2_body(agg2p, cnt_t, z2, out):
  s = agg2p[0] + agg2p[1]
  cnt = cnt_t[:, 0] + cnt_t[:, 1]
  inv = 1.0 / jnp.maximum(cnt, 1.0)
  out[...] = s * inv[:, None] + z2[...]


_tc2 = pl.pallas_call(
    _tc2_body,
    grid=(GRID,),
    in_specs=[
        pl.BlockSpec((NC, BLK, N_CLS), lambda i: (0, i, 0)),
        pl.BlockSpec((BLK, NC), lambda i: (i, 0)),
        pl.BlockSpec((BLK, N_CLS), lambda i: (i, 0)),
    ],
    out_specs=pl.BlockSpec((BLK, N_CLS), lambda i: (i, 0)),
    out_shape=jax.ShapeDtypeStruct((NP, N_CLS), jnp.float32),
)


def kernel(x, edge_index, W1_l, W1_r, b1, W2_l, W2_r, b2):
  src = edge_index[0].astype(jnp.int32)
  dst = edge_index[1].astype(jnp.int32)
  pad = EP - N_EDGES
  srcp = jnp.concatenate([src, jnp.zeros((pad,), jnp.int32)])
  dstp = jnp.concatenate([dst, jnp.full((pad,), NP - 1, jnp.int32)])
  srcp = srcp.reshape(NW, CH_PER_W, CHUNK)
  dstp = dstp.reshape(NW, CH_PER_W, CHUNK)

  aggp, cntp = _agg_l1(x, srcp, dstp)
  cnt_t = cntp.T
  xp = jnp.pad(x, ((0, NP - N_NODES), (0, 0)))

  h, y2, z2 = _tc1(aggp, cnt_t, xp, W1_l, W1_r, b1.reshape(1, -1),
                   W2_l, W2_r, b2.reshape(1, -1))

  (agg2p,) = _agg_l2(y2, srcp, dstp)
  out = _tc2(agg2p, cnt_t, z2)
  return out[:N_NODES]


# speedup vs baseline: 5.7743x; 1.2131x over previous
import functools

import jax
import jax.numpy as jnp
from jax import lax
from jax.experimental import pallas as pl
from jax.experimental.pallas import tpu as pltpu
from jax.experimental.pallas import tpu_sc as plsc

N_NODES = 10000
N_EDGES = 320000
D_IN = 128
D_HID = 128
N_CLS = 16

NC = 2
NS = 16
NW = NC * NS
CHUNK = 96
CH_PER_W = 106
EP = NW * CH_PER_W * CHUNK
NP = NS * 640
RPT = NP // NS

BLK = 128
GRID = NP // BLK


def _make_agg(d, with_cnt):
  out_type = [jax.ShapeDtypeStruct((NC, NP, d), jnp.float32)]
  if with_cnt:
    out_type.append(jax.ShapeDtypeStruct((NC, NP), jnp.float32))

  scratch = [
      pltpu.VMEM((CH_PER_W, CHUNK), jnp.int32),
      pltpu.VMEM((CH_PER_W, CHUNK), jnp.int32),
      pltpu.VMEM((CHUNK, d), jnp.float32),
      pltpu.VMEM((CHUNK, d), jnp.float32),
      pltpu.VMEM_SHARED((NP, d), jnp.float32),
      pltpu.SemaphoreType.DMA,
      pltpu.SemaphoreType.DMA,
  ]
  if with_cnt:
    scratch += [
        pltpu.VMEM((CHUNK,), jnp.float32),
        pltpu.VMEM((RPT,), jnp.float32),
        pltpu.VMEM_SHARED((NP,), jnp.float32),
        pltpu.SemaphoreType.DMA,
    ]

  mesh = plsc.VectorSubcoreMesh(core_axis_name="c", subcore_axis_name="s")

  @functools.partial(pl.kernel, mesh=mesh, out_type=out_type,
                     scratch_types=scratch,
                     compiler_params=pltpu.CompilerParams(
                         use_tc_tiling_on_sc=False))
  def body(feat_hbm, src_hbm, dst_hbm, *rest):
    if with_cnt:
      (agg_out, cnt_out, src_v, dst_v, rows0_v, rows1_v, agg_sh, sem0, sem1,
       ones_v, zc_v, cnt_sh, csem) = rest
    else:
      (agg_out, src_v, dst_v, rows0_v, rows1_v, agg_sh, sem0, sem1) = rest
    rows_v = rows0_v

    cid = lax.axis_index("c")
    sid = lax.axis_index("s")
    wid = cid * NS + sid

    pltpu.sync_copy(src_hbm.at[wid], src_v)
    pltpu.sync_copy(dst_hbm.at[wid], dst_v)

    z16 = jnp.zeros((16,), jnp.float32)
    g = d // 16

    def zrow(i, c):
      rows_v[i // g, pl.ds((i % g) * 16, 16)] = z16
      return c
    lax.fori_loop(0, CHUNK * g, zrow, 0)
    full, rem = divmod(RPT, CHUNK)
    for k in range(full):
      pltpu.sync_copy(rows_v,
                      agg_sh.at[pl.ds(sid * RPT + k * CHUNK, CHUNK)])
    if rem:
      pltpu.sync_copy(rows_v.at[pl.ds(0, rem)],
                      agg_sh.at[pl.ds(sid * RPT + full * CHUNK, rem)])

    if with_cnt:
      one16 = jnp.ones((16,), jnp.float32)
      for k in range(CHUNK // 16):
        ones_v[pl.ds(k * 16, 16)] = one16

      def zcnt(i, c):
        zc_v[pl.ds(i * 16, 16)] = z16
        return c
      lax.fori_loop(0, RPT // 16, zcnt, 0)
      pltpu.sync_copy(zc_v, cnt_sh.at[pl.ds(sid * RPT, RPT)])

    plsc.subcore_barrier()

    bufs = ((rows0_v, sem0), (rows1_v, sem1))
    pltpu.async_copy(feat_hbm.at[src_v.at[0]], rows0_v, sem0)

    def chunk2_body(j2, c):
      for b in (0, 1):
        jj = 2 * j2 + b
        rb, sb = bufs[b]
        ro, so = bufs[1 - b]
        pltpu.make_async_copy(feat_hbm.at[src_v.at[0]], rb, sb).wait()
        jn = lax.rem(jj + 1, CH_PER_W)
        pltpu.async_copy(feat_hbm.at[src_v.at[jn]], ro, so)
        pltpu.sync_copy(rb, agg_sh.at[dst_v.at[jj]], add=True)
        if with_cnt:
          pltpu.async_copy(ones_v, cnt_sh.at[dst_v.at[jj]], csem, add=True)
      return c
    lax.fori_loop(0, CH_PER_W // 2, chunk2_body, 0)

    pltpu.make_async_copy(feat_hbm.at[src_v.at[0]], rows0_v, sem0).wait()
    if with_cnt:
      def cnt_drain(j, c):
        pltpu.make_async_copy(ones_v, cnt_sh.at[dst_v.at[0]], csem).wait()
        return c
      lax.fori_loop(0, CH_PER_W, cnt_drain, 0)

    plsc.subcore_barrier()

    r0 = sid * RPT
    pltpu.sync_copy(agg_sh.at[pl.ds(r0, RPT)],
                    agg_out.at[cid, pl.ds(r0, RPT)])
    if with_cnt:
      pltpu.sync_copy(cnt_sh.at[pl.ds(r0, RPT)],
                      cnt_out.at[cid, pl.ds(r0, RPT)])

  return body


_agg_l1 = _make_agg(D_IN, with_cnt=True)
_agg_l2 = _make_agg(N_CLS, with_cnt=False)


def _tc1_body(aggp, cnt_t, xp, w1l, w1r, b1, w2l, w2r, b2,
              h_out, y2_out, z2_out):
  agg = aggp[0] + aggp[1]
  cnt = cnt_t[:, 0] + cnt_t[:, 1]
  inv = 1.0 / jnp.maximum(cnt, 1.0)
  mean = agg * inv[:, None]
  h = mean @ w1l[...] + xp[...] @ w1r[...] + b1[...]
  h = jnp.maximum(h, 0.0)
  h_out[...] = h
  y2_out[...] = h @ w2l[...]
  z2_out[...] = h @ w2r[...] + b2[...]


_tc1 = pl.pallas_call(
    _tc1_body,
    grid=(GRID,),
    in_specs=[
        pl.BlockSpec((NC, BLK, D_IN), lambda i: (0, i, 0)),
        pl.BlockSpec((BLK, NC), lambda i: (i, 0)),
        pl.BlockSpec((BLK, D_IN), lambda i: (i, 0)),
        pl.BlockSpec((D_IN, D_HID), lambda i: (0, 0)),
        pl.BlockSpec((D_IN, D_HID), lambda i: (0, 0)),
        pl.BlockSpec((1, D_HID), lambda i: (0, 0)),
        pl.BlockSpec((D_HID, N_CLS), lambda i: (0, 0)),
        pl.BlockSpec((D_HID, N_CLS), lambda i: (0, 0)),
        pl.BlockSpec((1, N_CLS), lambda i: (0, 0)),
    ],
    out_specs=[
        pl.BlockSpec((BLK, D_HID), lambda i: (i, 0)),
        pl.BlockSpec((BLK, N_CLS), lambda i: (i, 0)),
        pl.BlockSpec((BLK, N_CLS), lambda i: (i, 0)),
    ],
    out_shape=[
        jax.ShapeDtypeStruct((NP, D_HID), jnp.float32),
        jax.ShapeDtypeStruct((NP, N_CLS), jnp.float32),
        jax.ShapeDtypeStruct((NP, N_CLS), jnp.float32),
    ],
)


def _tc2_body(agg2p, cnt_t, z2, out):
  s = agg2p[0] + agg2p[1]
  cnt = cnt_t[:, 0] + cnt_t[:, 1]
  inv = 1.0 / jnp.maximum(cnt, 1.0)
  out[...] = s * inv[:, None] + z2[...]


_tc2 = pl.pallas_call(
    _tc2_body,
    grid=(GRID,),
    in_specs=[
        pl.BlockSpec((NC, BLK, N_CLS), lambda i: (0, i, 0)),
        pl.BlockSpec((BLK, NC), lambda i: (i, 0)),
        pl.BlockSpec((BLK, N_CLS), lambda i: (i, 0)),
    ],
    out_specs=pl.BlockSpec((BLK, N_CLS), lambda i: (i, 0)),
    out_shape=jax.ShapeDtypeStruct((NP, N_CLS), jnp.float32),
)


def kernel(x, edge_index, W1_l, W1_r, b1, W2_l, W2_r, b2):
  src = edge_index[0].astype(jnp.int32)
  dst = edge_index[1].astype(jnp.int32)
  pad = EP - N_EDGES
  srcp = jnp.concatenate([src, jnp.zeros((pad,), jnp.int32)])
  dstp = jnp.concatenate([dst, jnp.full((pad,), NP - 1, jnp.int32)])
  srcp = srcp.reshape(NW, CH_PER_W, CHUNK)
  dstp = dstp.reshape(NW, CH_PER_W, CHUNK)

  aggp, cntp = _agg_l1(x, srcp, dstp)
  cnt_t = cntp.T
  xp = jnp.pad(x, ((0, NP - N_NODES), (0, 0)))

  h, y2, z2 = _tc1(aggp, cnt_t, xp, W1_l, W1_r, b1.reshape(1, -1),
                   W2_l, W2_r, b2.reshape(1, -1))

  (agg2p,) = _agg_l2(y2, srcp, dstp)
  out = _tc2(agg2p, cnt_t, z2)
  return out[:N_NODES]


# speedup vs baseline: 8.7400x; 1.5136x over previous
import functools

import jax
import jax.numpy as jnp
from jax import lax
from jax.experimental import pallas as pl
from jax.experimental.pallas import tpu as pltpu
from jax.experimental.pallas import tpu_sc as plsc

N_NODES = 10000
N_EDGES = 320000
D_IN = 128
D_HID = 128
N_CLS = 16

NC = 2
NS = 16
NP = NS * 640
RPT = NP // NS

CHUNK1 = 64
CH1_C0 = 224
CH1_C1 = 90
EP1 = NS * (CH1_C0 + CH1_C1) * CHUNK1

CHUNK2 = 128
CH2_C0 = 84
CH2_C1 = 74
EP2 = NS * (CH2_C0 + CH2_C1) * CHUNK2


def _make_agg(d, with_cnt, chunk, ch0, ch1):
  ch_max = max(ch0, ch1)
  out_type = [jax.ShapeDtypeStruct((NC, NP, d), jnp.float32)]
  if with_cnt:
    out_type.append(jax.ShapeDtypeStruct((NC, NP), jnp.float32))

  scratch = [
      pltpu.VMEM((ch_max, chunk), jnp.int32),
      pltpu.VMEM((ch_max, chunk), jnp.int32),
      pltpu.VMEM((chunk, d), jnp.float32),
      pltpu.VMEM((chunk, d), jnp.float32),
      pltpu.VMEM_SHARED((NP, d), jnp.float32),
      pltpu.SemaphoreType.DMA,
      pltpu.SemaphoreType.DMA,
  ]
  if with_cnt:
    scratch += [
        pltpu.VMEM((chunk,), jnp.float32),
        pltpu.VMEM((RPT,), jnp.float32),
        pltpu.VMEM_SHARED((NP,), jnp.float32),
        pltpu.SemaphoreType.DMA,
    ]

  mesh = plsc.VectorSubcoreMesh(core_axis_name="c", subcore_axis_name="s")

  @functools.partial(pl.kernel, mesh=mesh, out_type=out_type,
                     scratch_types=scratch,
                     compiler_params=pltpu.CompilerParams(
                         use_tc_tiling_on_sc=False))
  def body(feat_hbm, src_hbm, dst_hbm, *rest):
    if with_cnt:
      (agg_out, cnt_out, src_v, dst_v, rows0_v, rows1_v, agg_sh, sem0, sem1,
       ones_v, zc_v, cnt_sh, csem) = rest
    else:
      (agg_out, src_v, dst_v, rows0_v, rows1_v, agg_sh, sem0, sem1) = rest

    cid = lax.axis_index("c")
    sid = lax.axis_index("s")
    wid = cid * NS + sid
    n_my = jnp.where(cid == 0, ch0, ch1)

    pltpu.sync_copy(src_hbm.at[wid], src_v)
    pltpu.sync_copy(dst_hbm.at[wid], dst_v)

    z16 = jnp.zeros((16,), jnp.float32)
    g = d // 16

    def zrow(i, c):
      rows0_v[i // g, pl.ds((i % g) * 16, 16)] = z16
      return c
    lax.fori_loop(0, chunk * g, zrow, 0)
    full, rem = divmod(RPT, chunk)
    for k in range(full):
      pltpu.sync_copy(rows0_v,
                      agg_sh.at[pl.ds(sid * RPT + k * chunk, chunk)])
    if rem:
      pltpu.sync_copy(rows0_v.at[pl.ds(0, rem)],
                      agg_sh.at[pl.ds(sid * RPT + full * chunk, rem)])

    if with_cnt:
      one16 = jnp.ones((16,), jnp.float32)
      for k in range(chunk // 16):
        ones_v[pl.ds(k * 16, 16)] = one16

      def zcnt(i, c):
        zc_v[pl.ds(i * 16, 16)] = z16
        return c
      lax.fori_loop(0, RPT // 16, zcnt, 0)
      pltpu.sync_copy(zc_v, cnt_sh.at[pl.ds(sid * RPT, RPT)])

    plsc.subcore_barrier()

    bufs = ((rows0_v, sem0), (rows1_v, sem1))
    pltpu.async_copy(feat_hbm.at[src_v.at[0]], rows0_v, sem0)

    def chunk2_body(j2, c):
      for b in (0, 1):
        jj = 2 * j2 + b
        rb, sb = bufs[b]
        ro, so = bufs[1 - b]
        pltpu.make_async_copy(feat_hbm.at[src_v.at[0]], rb, sb).wait()
        jn = lax.rem(jj + 1, n_my)
        pltpu.async_copy(feat_hbm.at[src_v.at[jn]], ro, so)
        pltpu.sync_copy(rb, agg_sh.at[dst_v.at[jj]], add=True)
        if with_cnt:
          pltpu.async_copy(ones_v, cnt_sh.at[dst_v.at[jj]], csem, add=True)
      return c
    lax.fori_loop(0, n_my // 2, chunk2_body, 0)

    pltpu.make_async_copy(feat_hbm.at[src_v.at[0]], rows0_v, sem0).wait()
    if with_cnt:
      def cnt_drain(j, c):
        pltpu.make_async_copy(ones_v, cnt_sh.at[dst_v.at[0]], csem).wait()
        return c
      lax.fori_loop(0, n_my, cnt_drain, 0)

    plsc.subcore_barrier()

    r0 = sid * RPT
    pltpu.sync_copy(agg_sh.at[pl.ds(r0, RPT)],
                    agg_out.at[cid, pl.ds(r0, RPT)])
    if with_cnt:
      pltpu.sync_copy(cnt_sh.at[pl.ds(r0, RPT)],
                      cnt_out.at[cid, pl.ds(r0, RPT)])

  return body


_agg_l1 = _make_agg(D_IN, True, CHUNK1, CH1_C0, CH1_C1)
_agg_l2 = _make_agg(N_CLS, False, CHUNK2, CH2_C0, CH2_C1)

BLK = 1024
GRID = NP // BLK


def _tc1_body(aggp, cnt_t, xp, w1l, w1r, b1, w2l, w2r, b2,
              h_out, y2_out, z2_out):
  agg = aggp[0] + aggp[1]
  cnt = cnt_t[:, 0] + cnt_t[:, 1]
  inv = 1.0 / jnp.maximum(cnt, 1.0)
  mean = agg * inv[:, None]
  h = mean @ w1l[...] + xp[...] @ w1r[...] + b1[...]
  h = jnp.maximum(h, 0.0)
  h_out[...] = h
  y2_out[...] = h @ w2l[...]
  z2_out[...] = h @ w2r[...] + b2[...]


_tc1 = pl.pallas_call(
    _tc1_body,
    grid=(GRID,),
    in_specs=[
        pl.BlockSpec((NC, BLK, D_IN), lambda i: (0, i, 0)),
        pl.BlockSpec((BLK, NC), lambda i: (i, 0)),
        pl.BlockSpec((BLK, D_IN), lambda i: (i, 0)),
        pl.BlockSpec((D_IN, D_HID), lambda i: (0, 0)),
        pl.BlockSpec((D_IN, D_HID), lambda i: (0, 0)),
        pl.BlockSpec((1, D_HID), lambda i: (0, 0)),
        pl.BlockSpec((D_HID, N_CLS), lambda i: (0, 0)),
        pl.BlockSpec((D_HID, N_CLS), lambda i: (0, 0)),
        pl.BlockSpec((1, N_CLS), lambda i: (0, 0)),
    ],
    out_specs=[
        pl.BlockSpec((BLK, D_HID), lambda i: (i, 0)),
        pl.BlockSpec((BLK, N_CLS), lambda i: (i, 0)),
        pl.BlockSpec((BLK, N_CLS), lambda i: (i, 0)),
    ],
    out_shape=[
        jax.ShapeDtypeStruct((NP, D_HID), jnp.float32),
        jax.ShapeDtypeStruct((NP, N_CLS), jnp.float32),
        jax.ShapeDtypeStruct((NP, N_CLS), jnp.float32),
    ],
)


def _tc2_body(agg2p, cnt_t, z2, out):
  s = agg2p[0] + agg2p[1]
  cnt = cnt_t[:, 0] + cnt_t[:, 1]
  inv = 1.0 / jnp.maximum(cnt, 1.0)
  out[...] = s * inv[:, None] + z2[...]


_tc2 = pl.pallas_call(
    _tc2_body,
    in_specs=[
        pl.BlockSpec((NC, NP, N_CLS), lambda: (0, 0, 0)),
        pl.BlockSpec((NP, NC), lambda: (0, 0)),
        pl.BlockSpec((NP, N_CLS), lambda: (0, 0)),
    ],
    out_specs=pl.BlockSpec((NP, N_CLS), lambda: (0, 0)),
    out_shape=jax.ShapeDtypeStruct((NP, N_CLS), jnp.float32),
)


def _edge_layout(src, dst, chunk, ch0, ch1):
  ch_max = max(ch0, ch1)
  ep = NS * (ch0 + ch1) * chunk
  pad = ep - N_EDGES
  srcp = jnp.concatenate([src, jnp.zeros((pad,), jnp.int32)])
  dstp = jnp.concatenate([dst, jnp.full((pad,), NP - 1, jnp.int32)])

  def split(e):
    e0 = e[:NS * ch0 * chunk].reshape(NS, ch0, chunk)
    e1 = e[NS * ch0 * chunk:].reshape(NS, ch1, chunk)
    e0 = jnp.pad(e0, ((0, 0), (0, ch_max - ch0), (0, 0)))
    e1 = jnp.pad(e1, ((0, 0), (0, ch_max - ch1), (0, 0)))
    return jnp.concatenate([e0, e1], axis=0)
  return split(srcp), split(dstp)


def kernel(x, edge_index, W1_l, W1_r, b1, W2_l, W2_r, b2):
  src = edge_index[0].astype(jnp.int32)
  dst = edge_index[1].astype(jnp.int32)
  src1, dst1 = _edge_layout(src, dst, CHUNK1, CH1_C0, CH1_C1)
  src2, dst2 = _edge_layout(src, dst, CHUNK2, CH2_C0, CH2_C1)

  aggp, cntp = _agg_l1(x, src1, dst1)
  cnt_t = cntp.T
  xp = jnp.pad(x, ((0, NP - N_NODES), (0, 0)))

  h, y2, z2 = _tc1(aggp, cnt_t, xp, W1_l, W1_r, b1.reshape(1, -1),
                   W2_l, W2_r, b2.reshape(1, -1))

  (agg2p,) = _agg_l2(y2, src2, dst2)
  out = _tc2(agg2p, cnt_t, z2)
  return out[:N_NODES]


# speedup vs baseline: 9.8967x; 1.1323x over previous
import functools

import jax
import jax.numpy as jnp
from jax import lax
from jax.experimental import pallas as pl
from jax.experimental.pallas import tpu as pltpu
from jax.experimental.pallas import tpu_sc as plsc

N_NODES = 10000
N_EDGES = 320000
D_IN = 128
D_HID = 128
N_CLS = 16

NC = 2
NS = 16
NP = NS * 640
RPT = NP // NS

CHUNK1 = 64
CH1_C0 = 198
CH1_C1 = 116
EP1 = NS * (CH1_C0 + CH1_C1) * CHUNK1

CHUNK2 = 128
CH2_C0 = 88
CH2_C1 = 70
EP2 = NS * (CH2_C0 + CH2_C1) * CHUNK2

EFLAT = 326784
M1 = EFLAT // CHUNK1
M2 = EFLAT // CHUNK2


def _make_agg(d, with_cnt, chunk, ch0, ch1):
  ch_max = max(ch0, ch1)
  out_type = [jax.ShapeDtypeStruct((NC, NP, d), jnp.float32)]
  if with_cnt:
    out_type.append(jax.ShapeDtypeStruct((NC, NP), jnp.float32))

  scratch = [
      pltpu.VMEM((ch_max, chunk), jnp.int32),
      pltpu.VMEM((ch_max, chunk), jnp.int32),
      pltpu.VMEM((chunk, d), jnp.float32),
      pltpu.VMEM((chunk, d), jnp.float32),
      pltpu.VMEM_SHARED((NP, d), jnp.float32),
      pltpu.SemaphoreType.DMA,
      pltpu.SemaphoreType.DMA,
  ]
  if with_cnt:
    scratch += [
        pltpu.VMEM((chunk,), jnp.float32),
        pltpu.VMEM((RPT,), jnp.float32),
        pltpu.VMEM_SHARED((NP,), jnp.float32),
        pltpu.SemaphoreType.DMA,
    ]

  mesh = plsc.VectorSubcoreMesh(core_axis_name="c", subcore_axis_name="s")

  @functools.partial(pl.kernel, mesh=mesh, out_type=out_type,
                     scratch_types=scratch,
                     compiler_params=pltpu.CompilerParams(
                         use_tc_tiling_on_sc=False))
  def body(feat_hbm, src_hbm, dst_hbm, *rest):
    if with_cnt:
      (agg_out, cnt_out, src_v, dst_v, rows0_v, rows1_v, agg_sh, sem0, sem1,
       ones_v, zc_v, cnt_sh, csem) = rest
    else:
      (agg_out, src_v, dst_v, rows0_v, rows1_v, agg_sh, sem0, sem1) = rest

    cid = lax.axis_index("c")
    sid = lax.axis_index("s")
    n_my = jnp.where(cid == 0, ch0, ch1)
    row0 = jnp.where(cid == 0, sid * ch0, NS * ch0 + sid * ch1)

    pltpu.sync_copy(src_hbm.at[pl.ds(row0, ch_max)], src_v)
    pltpu.sync_copy(dst_hbm.at[pl.ds(row0, ch_max)], dst_v)

    z16 = jnp.zeros((16,), jnp.float32)
    g = d // 16

    def zrow(i, c):
      rows0_v[i // g, pl.ds((i % g) * 16, 16)] = z16
      return c
    lax.fori_loop(0, chunk * g, zrow, 0)
    full, rem = divmod(RPT, chunk)
    for k in range(full):
      pltpu.sync_copy(rows0_v,
                      agg_sh.at[pl.ds(sid * RPT + k * chunk, chunk)])
    if rem:
      pltpu.sync_copy(rows0_v.at[pl.ds(0, rem)],
                      agg_sh.at[pl.ds(sid * RPT + full * chunk, rem)])

    if with_cnt:
      one16 = jnp.ones((16,), jnp.float32)
      for k in range(chunk // 16):
        ones_v[pl.ds(k * 16, 16)] = one16

      def zcnt(i, c):
        zc_v[pl.ds(i * 16, 16)] = z16
        return c
      lax.fori_loop(0, RPT // 16, zcnt, 0)
      pltpu.sync_copy(zc_v, cnt_sh.at[pl.ds(sid * RPT, RPT)])

    plsc.subcore_barrier()

    bufs = ((rows0_v, sem0), (rows1_v, sem1))
    pltpu.async_copy(feat_hbm.at[src_v.at[0]], rows0_v, sem0)

    def chunk2_body(j2, c):
      for b in (0, 1):
        jj = 2 * j2 + b
        rb, sb = bufs[b]
        ro, so = bufs[1 - b]
        pltpu.make_async_copy(feat_hbm.at[src_v.at[0]], rb, sb).wait()
        jn = lax.rem(jj + 1, n_my)
        pltpu.async_copy(feat_hbm.at[src_v.at[jn]], ro, so)
        pltpu.sync_copy(rb, agg_sh.at[dst_v.at[jj]], add=True)
        if with_cnt:
          pltpu.async_copy(ones_v, cnt_sh.at[dst_v.at[jj]], csem, add=True)
      return c
    lax.fori_loop(0, n_my // 2, chunk2_body, 0)

    pltpu.make_async_copy(feat_hbm.at[src_v.at[0]], rows0_v, sem0).wait()
    if with_cnt:
      def cnt_drain(j, c):
        pltpu.make_async_copy(ones_v, cnt_sh.at[dst_v.at[0]], csem).wait()
        return c
      lax.fori_loop(0, n_my, cnt_drain, 0)

    plsc.subcore_barrier()

    r0 = sid * RPT
    pltpu.sync_copy(agg_sh.at[pl.ds(r0, RPT)],
                    agg_out.at[cid, pl.ds(r0, RPT)])
    if with_cnt:
      pltpu.sync_copy(cnt_sh.at[pl.ds(r0, RPT)],
                      cnt_out.at[cid, pl.ds(r0, RPT)])

  return body


_agg_l1 = _make_agg(D_IN, True, CHUNK1, CH1_C0, CH1_C1)
_agg_l2 = _make_agg(N_CLS, False, CHUNK2, CH2_C0, CH2_C1)

BLK = 1024
GRID = NP // BLK


def _tc1_body(aggp, cnt_t, xp, w1l, w1r, b1, w2l, w2r, b2,
              h_out, y2_out, z2_out):
  agg = aggp[0] + aggp[1]
  cnt = cnt_t[:, 0] + cnt_t[:, 1]
  inv = 1.0 / jnp.maximum(cnt, 1.0)
  mean = agg * inv[:, None]
  h = mean @ w1l[...] + xp[...] @ w1r[...] + b1[...]
  h = jnp.maximum(h, 0.0)
  h_out[...] = h
  y2_out[...] = h @ w2l[...]
  z2_out[...] = h @ w2r[...] + b2[...]


_tc1 = pl.pallas_call(
    _tc1_body,
    grid=(GRID,),
    in_specs=[
        pl.BlockSpec((NC, BLK, D_IN), lambda i: (0, i, 0)),
        pl.BlockSpec((BLK, NC), lambda i: (i, 0)),
        pl.BlockSpec((BLK, D_IN), lambda i: (i, 0)),
        pl.BlockSpec((D_IN, D_HID), lambda i: (0, 0)),
        pl.BlockSpec((D_IN, D_HID), lambda i: (0, 0)),
        pl.BlockSpec((1, D_HID), lambda i: (0, 0)),
        pl.BlockSpec((D_HID, N_CLS), lambda i: (0, 0)),
        pl.BlockSpec((D_HID, N_CLS), lambda i: (0, 0)),
        pl.BlockSpec((1, N_CLS), lambda i: (0, 0)),
    ],
    out_specs=[
        pl.BlockSpec((BLK, D_HID), lambda i: (i, 0)),
        pl.BlockSpec((BLK, N_CLS), lambda i: (i, 0)),
        pl.BlockSpec((BLK, N_CLS), lambda i: (i, 0)),
    ],
    out_shape=[
        jax.ShapeDtypeStruct((NP, D_HID), jnp.float32),
        jax.ShapeDtypeStruct((NP, N_CLS), jnp.float32),
        jax.ShapeDtypeStruct((NP, N_CLS), jnp.float32),
    ],
)


def _tc2_body(agg2p, cnt_t, z2, out):
  s = agg2p[0] + agg2p[1]
  cnt = cnt_t[:, 0] + cnt_t[:, 1]
  inv = 1.0 / jnp.maximum(cnt, 1.0)
  out[...] = (s * inv[:, None] + z2[...])[:N_NODES]


_tc2 = pl.pallas_call(
    _tc2_body,
    in_specs=[
        pl.BlockSpec((NC, NP, N_CLS), lambda: (0, 0, 0)),
        pl.BlockSpec((NP, NC), lambda: (0, 0)),
        pl.BlockSpec((NP, N_CLS), lambda: (0, 0)),
    ],
    out_specs=pl.BlockSpec((N_NODES, N_CLS), lambda: (0, 0)),
    out_shape=jax.ShapeDtypeStruct((N_NODES, N_CLS), jnp.float32),
)


def kernel(x, edge_index, W1_l, W1_r, b1, W2_l, W2_r, b2):
  src = edge_index[0].astype(jnp.int32)
  dst = edge_index[1].astype(jnp.int32)
  pad = EFLAT - N_EDGES
  srcp = jnp.concatenate([src, jnp.zeros((pad,), jnp.int32)])
  dstp = jnp.concatenate([dst, jnp.full((pad,), NP - 1, jnp.int32)])

  aggp, cntp = _agg_l1(x, srcp.reshape(M1, CHUNK1), dstp.reshape(M1, CHUNK1))
  cnt_t = cntp.T
  xp = jnp.pad(x, ((0, NP - N_NODES), (0, 0)))

  h, y2, z2 = _tc1(aggp, cnt_t, xp, W1_l, W1_r, b1.reshape(1, -1),
                   W2_l, W2_r, b2.reshape(1, -1))

  (agg2p,) = _agg_l2(y2, srcp.reshape(M2, CHUNK2), dstp.reshape(M2, CHUNK2))
  return _tc2(agg2p, cnt_t, z2)
